# Initial kernel scaffold; baseline (speedup 1.0000x reference)
import jax
import jax.numpy as jnp
from jax.experimental import pallas as pl


def kernel(x, edge_index, edge_attr, edge_weight, batch, intermediate_node_emb, params):
    raise NotImplementedError("write your pallas kernel here")


# speedup vs baseline: 3.7582x; 3.7582x over previous
import functools

import jax
import jax.numpy as jnp
from jax import lax
from jax.experimental import pallas as pl
from jax.experimental.pallas import tpu as pltpu
from jax.experimental.pallas import tpu_sc as plsc

F32 = jnp.float32
I32 = jnp.int32

NC = 2
NS = 16
NW = NC * NS
HP = 128


def _embed_body(x_ref, it_ref, aw, ab, ew, eb, mw, mb, out_ref):
    a = jnp.dot(x_ref[...], aw[...], preferred_element_type=F32) + ab[...]
    e = jnp.dot(it_ref[...], ew[...], preferred_element_type=F32) + eb[...]
    m = mw[...]
    h = jnp.dot(a, m[: a.shape[1]], preferred_element_type=F32)
    h = h + jnp.dot(e, m[a.shape[1]:], preferred_element_type=F32) + mb[...]
    h = jnp.maximum(h, 0.0)
    out_ref[...] = jnp.pad(h, ((0, 0), (0, HP - h.shape[1])))


def _embed(x, inter, aw, ab, ew, eb, mw, mb):
    n = x.shape[0]
    return pl.pallas_call(
        _embed_body,
        out_shape=jax.ShapeDtypeStruct((n, HP), F32),
    )(x, inter, aw, ab, ew, eb, mw, mb)


def _edge_mlp_body(nl, ea_ref, w1, b1, w2, b2, *outs):
    ea = ea_ref[...]
    w1v = w1[...]
    b1v = b1[...]
    w2v = w2[...]
    b2v = b2[...]
    for l in range(nl):
        t = jnp.maximum(jnp.dot(ea, w1v[l], preferred_element_type=F32) + b1v[l], 0.0)
        outs[l][...] = jnp.dot(t, w2v[l], preferred_element_type=F32) + b2v[l]


def _edge_mlp(edge_attr, w1s, b1s, w2s, b2s):
    e, ef = edge_attr.shape
    nl, _, h = w1s.shape
    bl = 10000
    grid = e // bl
    return pl.pallas_call(
        functools.partial(_edge_mlp_body, nl),
        grid=(grid,),
        in_specs=[
            pl.BlockSpec((bl, ef), lambda i: (i, 0)),
            pl.BlockSpec((nl, ef, h), lambda i: (0, 0, 0)),
            pl.BlockSpec((nl, h), lambda i: (0, 0)),
            pl.BlockSpec((nl, h, h), lambda i: (0, 0, 0)),
            pl.BlockSpec((nl, h), lambda i: (0, 0)),
        ],
        out_specs=[pl.BlockSpec((bl, h), lambda i: (i, 0)) for _ in range(nl)],
        out_shape=[jax.ShapeDtypeStruct((e, h), F32) for _ in range(nl)],
    )(edge_attr, w1s, b1s, w2s, b2s)


def _sc_edge_body(N, H, PW, CH, n_full, tail,
                  h_hbm, e_hbm, src_hbm, dst_hbm, ew_hbm, z_hbm, out_hbm,
                  idx_s, idx_d, ewb, hrows, erows, zbuf,
                  idx_st, idx_dt, ewbt, aggsh, sem):
    c = lax.axis_index("c")
    s = lax.axis_index("s")
    wid = s * NC + c
    zrows = zbuf.shape[0]
    rows_per_sub = N // NS

    pltpu.sync_copy(z_hbm, zbuf)
    for j in range(rows_per_sub // zrows):
        pltpu.sync_copy(zbuf, aggsh.at[pl.ds(s * rows_per_sub + j * zrows, zrows)])
    plsc.subcore_barrier()

    base = wid * PW

    def compute_rows(nrows, ew_ref, hr, er):
        dn = lax.GatherDimensionNumbers(offset_dims=(), collapsed_slice_dims=(0,),
                                        start_index_map=(0,))

        def grp(b, carry):
            ewv = ew_ref[pl.ds(b * 16, 16)]
            for j in range(16):
                r = b * 16 + j
                spl = lax.gather(ewv, jnp.full((16, 1), j, I32), dn,
                                 slice_sizes=(1,),
                                 mode=lax.GatherScatterMode.PROMISE_IN_BOUNDS)
                for k in range(H // 16):
                    hv = hr[r, pl.ds(k * 16, 16)]
                    ev = er[r, pl.ds(k * 16, 16)]
                    hr[r, pl.ds(k * 16, 16)] = jnp.maximum(hv + ev, 0.0) * spl
            return carry

        lax.fori_loop(0, nrows // 16, grp, 0)

    def chunk(ch, carry):
        off = base + ch * CH
        pltpu.sync_copy(src_hbm.at[pl.ds(off, CH)], idx_s)
        pltpu.sync_copy(dst_hbm.at[pl.ds(off, CH)], idx_d)
        pltpu.sync_copy(ew_hbm.at[pl.ds(off, CH)], ewb)
        pltpu.async_copy(h_hbm.at[idx_s], hrows, sem).wait()
        pltpu.sync_copy(e_hbm.at[pl.ds(off, CH)], erows)
        compute_rows(CH, ewb, hrows, erows)
        pltpu.sync_copy(hrows, aggsh.at[idx_d], add=True)
        return carry

    lax.fori_loop(0, n_full, chunk, 0)

    if tail:
        offt = base + n_full * CH
        pltpu.sync_copy(src_hbm.at[pl.ds(offt, tail)], idx_st)
        pltpu.sync_copy(dst_hbm.at[pl.ds(offt, tail)], idx_dt)
        pltpu.sync_copy(ew_hbm.at[pl.ds(offt, tail)], ewbt)
        pltpu.async_copy(h_hbm.at[idx_st], hrows.at[pl.ds(0, tail)], sem).wait()
        pltpu.sync_copy(e_hbm.at[pl.ds(offt, tail)], erows.at[pl.ds(0, tail)])
        compute_rows(tail, ewbt, hrows, erows)
        pltpu.sync_copy(hrows.at[pl.ds(0, tail)], aggsh.at[idx_dt], add=True)

    plsc.subcore_barrier()
    slab = (N // NS) & ~7
    rem = N - slab * NS
    pltpu.sync_copy(aggsh.at[pl.ds(s * slab, slab)],
                    out_hbm.at[c, pl.ds(s * slab, slab)])
    if rem:
        @pl.when(s == NS - 1)
        def _():
            pltpu.sync_copy(aggsh.at[pl.ds(slab * NS, rem)],
                            out_hbm.at[c, pl.ds(slab * NS, rem)])


def _sc_edge(h, e, src, dst, ew, zin):
    N = h.shape[0]
    H = e.shape[1]
    E = src.shape[0]
    PW = E // NW
    CH = 128
    n_full = PW // CH
    tail = PW - n_full * CH
    mesh = plsc.VectorSubcoreMesh(core_axis_name="c", subcore_axis_name="s",
                                  num_cores=NC, num_subcores=NS)
    body = functools.partial(_sc_edge_body, N, H, PW, CH, n_full, tail)
    fn = pl.kernel(
        body,
        out_type=jax.ShapeDtypeStruct((NC, N, HP), F32),
        mesh=mesh,
        scratch_types=[
            pltpu.VMEM((CH,), I32),
            pltpu.VMEM((CH,), I32),
            pltpu.VMEM((CH,), F32),
            pltpu.VMEM((CH, HP), F32),
            pltpu.VMEM((CH, H), F32),
            pltpu.VMEM((125, HP), F32),
            pltpu.VMEM((max(tail, 16),), I32),
            pltpu.VMEM((max(tail, 16),), I32),
            pltpu.VMEM((max(tail, 16),), F32),
            pltpu.VMEM_SHARED((N, HP), F32),
            pltpu.SemaphoreType.DMA,
        ],
    )
    return fn(h, e, src, dst, ew, zin)


def _node_body(first, hd, h_ref, agg_ref, w1, b1, w2, b2, g, bt, eps_ref, out_ref):
    h = h_ref[...][:, :hd]
    z = (1.0 + eps_ref[0, 0]) * h + agg_ref[0, :, :hd] + agg_ref[1, :, :hd]
    z = jnp.maximum(jnp.dot(z, w1[...], preferred_element_type=F32) + b1[...], 0.0)
    z = jnp.dot(z, w2[...], preferred_element_type=F32) + b2[...]
    m = jnp.mean(z, axis=0, keepdims=True)
    d = z - m
    v = jnp.mean(d * d, axis=0, keepdims=True)
    hn = jnp.maximum(d * lax.rsqrt(v + 1e-5) * g[...] + bt[...], 0.0)
    hn = hn if first else h + hn
    out_ref[...] = jnp.pad(hn, ((0, 0), (0, HP - hd)))


def _node(h, aggp, w1, b1, w2, b2, g, bt, eps, first):
    n = h.shape[0]
    hd = w1.shape[0]
    return pl.pallas_call(
        functools.partial(_node_body, first, hd),
        out_shape=jax.ShapeDtypeStruct((n, HP), F32),
    )(h, aggp, w1, b1, w2, b2, g, bt, eps)


def _sc_pool_body(N, G, CH, n_full, tail,
                  h_hbm, b_hbm, z_hbm, o_hbm, sum_hbm, cnt_hbm,
                  ibuf, hbuf, obuf, zbuf, ibt, ssh, csh, sem):
    c = lax.axis_index("c")
    s = lax.axis_index("s")
    wid = s * NC + c
    gps = G // NS

    pltpu.sync_copy(z_hbm, zbuf)
    pltpu.sync_copy(o_hbm, obuf)
    pltpu.sync_copy(zbuf.at[pl.ds(0, gps)], ssh.at[pl.ds(s * gps, gps)])
    pltpu.sync_copy(zbuf.at[pl.ds(0, gps)], csh.at[pl.ds(s * gps, gps)])
    plsc.subcore_barrier()

    n_rounds = (n_full + NW - 1) // NW
    for t in range(n_rounds):
        ch = wid + t * NW

        @pl.when(ch < n_full)
        def _():
            off = ch * CH
            pltpu.sync_copy(b_hbm.at[pl.ds(off, CH)], ibuf)
            pltpu.sync_copy(h_hbm.at[pl.ds(off, CH)], hbuf)
            pltpu.sync_copy(hbuf, ssh.at[ibuf], add=True)
            pltpu.sync_copy(obuf, csh.at[ibuf], add=True)

    if tail:
        @pl.when(wid == 0)
        def _():
            offt = n_full * CH
            pltpu.sync_copy(b_hbm.at[pl.ds(offt, tail)], ibt)
            pltpu.sync_copy(h_hbm.at[pl.ds(offt, tail)], hbuf.at[pl.ds(0, tail)])
            pltpu.sync_copy(hbuf.at[pl.ds(0, tail)], ssh.at[ibt], add=True)
            pltpu.sync_copy(obuf.at[pl.ds(0, tail)], csh.at[ibt], add=True)

    plsc.subcore_barrier()
    rs = s * gps
    pltpu.sync_copy(ssh.at[pl.ds(rs, gps)], sum_hbm.at[c, pl.ds(rs, gps)])
    pltpu.sync_copy(csh.at[pl.ds(rs, gps)], cnt_hbm.at[c, pl.ds(rs, gps)])


def _sc_pool(h, batch, G, zin, oin):
    N = h.shape[0]
    CH = 128
    n_full = N // CH
    tail = N - n_full * CH
    mesh = plsc.VectorSubcoreMesh(core_axis_name="c", subcore_axis_name="s",
                                  num_cores=NC, num_subcores=NS)
    body = functools.partial(_sc_pool_body, N, G, CH, n_full, tail)
    fn = pl.kernel(
        body,
        out_type=[jax.ShapeDtypeStruct((NC, G, HP), F32),
                  jax.ShapeDtypeStruct((NC, G, HP), F32)],
        mesh=mesh,
        scratch_types=[
            pltpu.VMEM((CH,), I32),
            pltpu.VMEM((CH, HP), F32),
            pltpu.VMEM((CH, HP), F32),
            pltpu.VMEM((125, HP), F32),
            pltpu.VMEM((max(tail, 16),), I32),
            pltpu.VMEM_SHARED((G, HP), F32),
            pltpu.VMEM_SHARED((G, HP), F32),
            pltpu.SemaphoreType.DMA,
        ],
    )
    return fn(h, batch, zin, oin)


def _final_body(hd, sp, cp, w1, b1, w2, b2, w3, b3, out_ref):
    s_ = sp[...]
    c_ = cp[...]
    sums = s_[0, :, :hd] + s_[1, :, :hd]
    counts = c_[0, :, 0:1] + c_[1, :, 0:1]
    hg = sums / jnp.maximum(counts, 1.0)
    hg = jnp.maximum(jnp.dot(hg, w1[...], preferred_element_type=F32) + b1[...], 0.0)
    hg = jnp.maximum(jnp.dot(hg, w2[...], preferred_element_type=F32) + b2[...], 0.0)
    out_ref[...] = jnp.dot(hg, w3[...], preferred_element_type=F32) + b3[...]


def _final(sp, cp, w1, b1, w2, b2, w3, b3):
    g = sp.shape[1]
    hd = w1.shape[0]
    return pl.pallas_call(
        functools.partial(_final_body, hd),
        out_shape=jax.ShapeDtypeStruct((g, w3.shape[1]), F32),
    )(sp, cp, w1, b1, w2, b2, w3, b3)


def kernel(x, edge_index, edge_attr, edge_weight, batch, intermediate_node_emb,
           params):
    src = edge_index[0].astype(I32)
    dst = edge_index[1].astype(I32)
    batch32 = batch.astype(I32)
    ew = edge_weight.astype(F32)
    G = 512

    p = params
    convs = p["convs"]

    zin = jnp.zeros((125, HP), F32)
    oin = jnp.ones((128, HP), F32)

    h = _embed(x, intermediate_node_emb,
               p["atom_W"], p["atom_b"].reshape(1, -1),
               p["extra_W"], p["extra_b"].reshape(1, -1),
               p["merge_W"], p["merge_b"].reshape(1, -1))

    w1s = jnp.stack([c["be_W1"] for c in convs])
    b1s = jnp.stack([c["be_b1"] for c in convs])
    w2s = jnp.stack([c["be_W2"] for c in convs])
    b2s = jnp.stack([c["be_b2"] for c in convs])
    es = _edge_mlp(edge_attr, w1s, b1s, w2s, b2s)

    for l, c in enumerate(convs):
        aggp = _sc_edge(h, es[l], src, dst, ew, zin)
        h = _node(h, aggp, c["W1"], c["b1"].reshape(1, -1),
                  c["W2"], c["b2"].reshape(1, -1),
                  c["gamma"].reshape(1, -1), c["beta"].reshape(1, -1),
                  c["eps"].reshape(1, 1), first=(l == 0))

    sp, cp = _sc_pool(h, batch32, G, zin, oin)
    return _final(sp, cp,
                  p["mlp_W1"], p["mlp_b1"].reshape(1, -1),
                  p["mlp_W2"], p["mlp_b2"].reshape(1, -1),
                  p["mlp_W3"], p["mlp_b3"].reshape(1, -1))
